# baseline (speedup 1.0000x reference)
import functools
import math

import jax
import jax.numpy as jnp
from jax.experimental import pallas as pl
from jax.experimental.pallas import tpu as pltpu

_T = math.e
_AR = 0.7


def _body(x_ref, w_ref, b_ref, o_ref, *, na):
    l = jax.lax.dot_general(
        x_ref[...], w_ref[...],
        (((1,), (1,)), ((), ())),
        preferred_element_type=jnp.float32,
        precision=jax.lax.Precision.DEFAULT,
    )
    scaled = (l + b_ref[...]) * (1.0 / _T)
    m = jnp.max(scaled, axis=1, keepdims=True)
    e = jnp.exp(scaled - m)

    bmr = e.shape[0]
    n = e.shape[1]

    def rowsum(v):
        return jnp.sum(v, axis=1, keepdims=True)

    esum = rowsum(e)

    key = jax.lax.bitcast_convert_type(e, jnp.int32)
    fna = jnp.float32(na)

    def cond(st):
        _, lo, hi, c_lo, c_hi = st
        return jnp.any((c_lo != fna) & (hi - lo > 1) & (c_lo - c_hi > 2.0))

    mu = esum * (1.0 / n)

    def body(st):
        it, lo, hi, c_lo, c_hi = st
        frac = (fna - c_hi) / jnp.maximum(c_lo - c_hi, 1.0)
        width = hi - lo
        t_key = hi - (width.astype(jnp.float32) * frac).astype(jnp.int32)
        vlo = jax.lax.bitcast_convert_type(lo, jnp.float32)
        vhi = jax.lax.bitcast_convert_type(hi, jnp.float32)
        t_flt = jax.lax.bitcast_convert_type(vhi - (vhi - vlo) * frac,
                                             jnp.int32)
        t = jnp.where(width <= (1 << 16), t_key, t_flt)
        t = jnp.where(it % 8 == 7, lo + (width >> 1), t)
        w1 = jax.lax.bitcast_convert_type(0.875 * mu, jnp.int32)
        w2 = jax.lax.bitcast_convert_type(
            jnp.where(c_lo == jnp.float32(n), 0.860, 0.890) * mu, jnp.int32)
        t = jnp.where(it == 0, w1, jnp.where(it == 1, w2, t))
        t = jnp.clip(t, lo + 1, hi - 1)
        cnt = rowsum((key >= t).astype(jnp.float32))
        ge = cnt >= fna
        return (it + 1,
                jnp.where(ge, t, lo), jnp.where(ge, hi, t),
                jnp.where(ge, cnt, c_lo), jnp.where(ge, c_hi, cnt))

    init = (jnp.int32(0),
            jnp.zeros((bmr, 1), jnp.int32),
            jnp.full((bmr, 1), 0x3F800001, jnp.int32),
            jnp.full((bmr, 1), float(n), jnp.float32),
            jnp.zeros((bmr, 1), jnp.float32))
    _, lo, hi, c_lo, c_hi = jax.lax.while_loop(cond, body, init)

    def braket_max(ub):
        inb = (key >= lo) & (key < ub)
        return jnp.max(jnp.where(inb, key, 0), axis=1, keepdims=True)

    v1 = braket_max(hi)
    v2 = braket_max(v1)
    m = fna - c_hi
    t_ext = jnp.maximum(jnp.where(m == 1.0, v1, v2), lo)
    thr = jnp.where((c_lo == fna) | (hi - lo == 1), lo, t_ext)

    d = jnp.where(key >= thr, e, 0.0)
    s = rowsum(d)
    recip = 1.0 / (s + 1e-8 * esum)
    o_ref[...] = d * recip


def kernel(x, W, b):
    rows, d = x.shape
    n = W.shape[0]
    na = max(1, int(n * _AR))
    bm = 128
    while rows % bm:
        bm //= 2
    grid = (rows // bm,)
    b2 = b.reshape(1, n)
    return pl.pallas_call(
        functools.partial(_body, na=na),
        grid=grid,
        in_specs=[
            pl.BlockSpec((bm, d), lambda i: (i, 0)),
            pl.BlockSpec((n, d), lambda i: (0, 0)),
            pl.BlockSpec((1, n), lambda i: (0, 0)),
        ],
        out_specs=pl.BlockSpec((bm, n), lambda i: (i, 0)),
        out_shape=jax.ShapeDtypeStruct((rows, n), jnp.float32),
        compiler_params=pltpu.CompilerParams(
            dimension_semantics=("parallel",),
        ),
    )(x, W, b2)


# speedup vs baseline: 1.5084x; 1.5084x over previous
import functools
import math

import jax
import jax.numpy as jnp
from jax.experimental import pallas as pl
from jax.experimental.pallas import tpu as pltpu

_T = math.e
_AR = 0.7


def _body(x_ref, w_ref, b_ref, o_ref, *, na):
    l = jax.lax.dot_general(
        x_ref[...], w_ref[...],
        (((1,), (1,)), ((), ())),
        preferred_element_type=jnp.float32,
        precision=jax.lax.Precision.DEFAULT,
    )
    scaled = (l + b_ref[...]) * (1.0 / _T)
    m = jnp.max(scaled, axis=1, keepdims=True)
    e = jnp.exp(scaled - m)

    bmr = e.shape[0]
    n = e.shape[1]

    def rowsum(v):
        return jnp.sum(v, axis=1, keepdims=True)

    esum = rowsum(e)

    key = jax.lax.bitcast_convert_type(e, jnp.int32)
    fna = jnp.float32(na)

    def cond(st):
        _, lo, hi, c_lo, c_hi = st
        return jnp.any((c_lo != fna) & (hi - lo > 1) & (c_lo - c_hi > 2.0))

    mu = esum * (1.0 / n)

    def body(st):
        it, lo, hi, c_lo, c_hi = st
        frac = (fna - c_hi) / jnp.maximum(c_lo - c_hi, 1.0)
        width = hi - lo
        t_key = hi - (width.astype(jnp.float32) * frac).astype(jnp.int32)
        vlo = jax.lax.bitcast_convert_type(lo, jnp.float32)
        vhi = jax.lax.bitcast_convert_type(hi, jnp.float32)
        t_flt = jax.lax.bitcast_convert_type(vhi - (vhi - vlo) * frac,
                                             jnp.int32)
        t = jnp.where(width <= (1 << 16), t_key, t_flt)
        t = jnp.where(it % 8 == 7, lo + (width >> 1), t)
        w1 = jax.lax.bitcast_convert_type(0.875 * mu, jnp.int32)
        w2 = jax.lax.bitcast_convert_type(
            jnp.where(c_lo == jnp.float32(n), 0.860, 0.890) * mu, jnp.int32)
        t = jnp.where(it == 0, w1, jnp.where(it == 1, w2, t))
        t = jnp.clip(t, lo + 1, hi - 1)
        cnt = rowsum((key >= t).astype(jnp.float32))
        ge = cnt >= fna
        return (it + 1,
                jnp.where(ge, t, lo), jnp.where(ge, hi, t),
                jnp.where(ge, cnt, c_lo), jnp.where(ge, c_hi, cnt))

    init = (jnp.int32(0),
            jnp.zeros((bmr, 1), jnp.int32),
            jnp.full((bmr, 1), 0x3F800001, jnp.int32),
            jnp.full((bmr, 1), float(n), jnp.float32),
            jnp.zeros((bmr, 1), jnp.float32))
    _, lo, hi, c_lo, c_hi = jax.lax.while_loop(cond, body, init)

    def braket_max(ub):
        inb = (key >= lo) & (key < ub)
        return jnp.max(jnp.where(inb, key, 0), axis=1, keepdims=True)

    v1 = braket_max(hi)
    v2 = braket_max(v1)
    m = fna - c_hi
    t_ext = jnp.maximum(jnp.where(m == 1.0, v1, v2), lo)
    thr = jnp.where((c_lo == fna) | (hi - lo == 1), lo, t_ext)

    d = jnp.where(key >= thr, e, 0.0)
    s = rowsum(d)
    recip = 1.0 / (s + 1e-8 * esum)
    o_ref[...] = d * recip


def kernel(x, W, b):
    rows, d = x.shape
    n = W.shape[0]
    na = max(1, int(n * _AR))
    bm = 256
    while rows % bm:
        bm //= 2
    grid = (rows // bm,)
    b2 = b.reshape(1, n)
    return pl.pallas_call(
        functools.partial(_body, na=na),
        grid=grid,
        in_specs=[
            pl.BlockSpec((bm, d), lambda i: (i, 0)),
            pl.BlockSpec((n, d), lambda i: (0, 0)),
            pl.BlockSpec((1, n), lambda i: (0, 0)),
        ],
        out_specs=pl.BlockSpec((bm, n), lambda i: (i, 0)),
        out_shape=jax.ShapeDtypeStruct((rows, n), jnp.float32),
        compiler_params=pltpu.CompilerParams(
            dimension_semantics=("parallel",),
        ),
    )(x, W, b2)
